# baseline (speedup 1.0000x reference)
import functools

import jax
import jax.numpy as jnp
from jax import lax
from jax.experimental import pallas as pl
from jax.experimental.pallas import tpu as pltpu
from jax.experimental.pallas import tpu_sc as plsc

E = 8
D = 2048
FF = 1408
SFF = 2816
T = 2048
K = 2
BT = 256
NT = 23
PP = NT * BT
NC = 2
NS = 16
NW = NC * NS
RW = PP // NW
TW = T // NW
TBR = 128


def _router_body(x_ref, g_ref, ints_ref, ws_ref, cnt_ref):
    i = pl.program_id(0)

    @pl.when(i == 0)
    def _():
        cnt_ref[...] = jnp.zeros_like(cnt_ref)

    x = x_ref[...]
    logits = lax.dot_general(
        x, g_ref[...], (((1,), (1,)), ((), ())),
        preferred_element_type=jnp.float32)

    tri = (lax.broadcasted_iota(jnp.int32, (E, E), 0)
           <= lax.broadcasted_iota(jnp.int32, (E, E), 1)).astype(jnp.float32)
    iota_e = lax.broadcasted_iota(jnp.int32, (TBR, E), 1).astype(jnp.float32)

    def pick(m):
        r = jnp.max(m, axis=1, keepdims=True)
        hit = (m == r).astype(jnp.float32)
        csum = lax.dot_general(hit, tri, (((1,), (0,)), ((), ())),
                               preferred_element_type=jnp.float32)
        first = hit * (csum == 1.0).astype(jnp.float32)
        e = jnp.sum(first * iota_e, axis=1, keepdims=True)
        return r, first, e

    r1, f1, e1 = pick(logits)
    r2, f2, e2 = pick(logits - f1 * 1e30)
    w_a = 1.0 / (1.0 + jnp.exp(r2 - r1))
    w_b = 1.0 / (1.0 + jnp.exp(r1 - r2))

    cnt = f1 + f2
    lower = (lax.broadcasted_iota(jnp.int32, (TBR, TBR), 1)
             < lax.broadcasted_iota(jnp.int32, (TBR, TBR), 0)).astype(jnp.float32)
    pos_in = lax.dot_general(lower, cnt, (((1,), (0,)), ((), ())),
                             preferred_element_type=jnp.float32)
    prev = cnt_ref[...]
    pos = pos_in + prev
    p1 = jnp.sum(f1 * pos, axis=1, keepdims=True)
    p2 = jnp.sum(f2 * pos, axis=1, keepdims=True)
    cnt_ref[...] = prev + jnp.sum(cnt, axis=0, keepdims=True)

    col4 = lax.broadcasted_iota(jnp.int32, (TBR, 4), 1)
    iv = jnp.where(col4 == 0, e1,
                   jnp.where(col4 == 1, e2, jnp.where(col4 == 2, p1, p2)))
    ints_ref[...] = iv.astype(jnp.int32)
    col2 = lax.broadcasted_iota(jnp.int32, (TBR, 2), 1)
    ws_ref[...] = jnp.where(col2 == 0, w_a, w_b)


_router = pl.pallas_call(
    _router_body,
    grid=(T // TBR,),
    in_specs=[
        pl.BlockSpec((TBR, D), lambda i: (i, 0)),
        pl.BlockSpec((E, D), lambda i: (0, 0)),
    ],
    out_specs=[
        pl.BlockSpec((TBR, 4), lambda i: (i, 0)),
        pl.BlockSpec((TBR, 2), lambda i: (i, 0)),
        pl.BlockSpec((1, E), lambda i: (0, 0)),
    ],
    out_shape=[
        jax.ShapeDtypeStruct((T, 4), jnp.int32),
        jax.ShapeDtypeStruct((T, 2), jnp.float32),
        jax.ShapeDtypeStruct((1, E), jnp.float32),
    ],
)


_CHUNKS = [(0, 24), (24, 24), (48, 24), (72, 24),
           (96, 24), (120, 24), (144, 24), (168, 16)]


def _dispatch_body(ints_hbm, ws_hbm, base_hbm, x_hbm,
                   xs_hbm, wso_hbm, slot_hbm,
                   ints_v, ws_v, base_v, tok_v, wv, slots_v, rows_v, sem):
    wid = lax.axis_index("s") * NC + lax.axis_index("c")
    pltpu.sync_copy(ints_hbm, ints_v)
    pltpu.sync_copy(ws_hbm, ws_v)
    pltpu.sync_copy(base_hbm, base_v)

    def zero_body(j, _):
        tok_v[pl.ds(j * 16, 16)] = jnp.zeros((16,), jnp.int32)
        wv[pl.ds(j * 16, 16)] = jnp.zeros((16,), jnp.float32)
        return 0

    lax.fori_loop(0, PP // 16, zero_body, 0)

    def scat_body(c, _):
        t0 = c * 16
        tok = lax.iota(jnp.int32, 16) + t0
        for k in range(K):
            e = ints_v[k, pl.ds(t0, 16)]
            p = ints_v[K + k, pl.ds(t0, 16)]
            w = ws_v[k, pl.ds(t0, 16)]
            slot = plsc.load_gather(base_v, [e]) + p
            plsc.store_scatter(tok_v, [slot], tok)
            plsc.store_scatter(wv, [slot], w)
            slots_v[pl.ds(k * T + t0, 16)] = slot
        return 0

    lax.fori_loop(0, T // 16, scat_body, 0)

    pltpu.sync_copy(wv.at[pl.ds(wid * RW, RW)], wso_hbm.at[pl.ds(wid * RW, RW)])

    @pl.when(wid == 0)
    def _():
        pltpu.sync_copy(slots_v, slot_hbm)

    for off, n in _CHUNKS:
        idx = tok_v.at[pl.ds(wid * RW + off, n)]
        pltpu.async_copy(x_hbm.at[idx], rows_v.at[pl.ds(0, n)], sem).wait()
        pltpu.sync_copy(rows_v.at[pl.ds(0, n)],
                        xs_hbm.at[pl.ds(wid * RW + off, n)])


def _gmm_body(sp_ref, xs_ref, w1g_ref, w1u_ref, w2_ref, ws_ref, y_ref):
    i = pl.program_id(0)

    @pl.when(i < sp_ref[NT])
    def _():
        a = xs_ref[...]
        g = lax.dot_general(a, w1g_ref[0], (((1,), (1,)), ((), ())),
                            preferred_element_type=jnp.float32)
        u = lax.dot_general(a, w1u_ref[0], (((1,), (1,)), ((), ())),
                            preferred_element_type=jnp.float32)
        act = (g * jax.nn.sigmoid(g) * u).astype(jnp.bfloat16)
        y = lax.dot_general(act, w2_ref[0], (((1,), (1,)), ((), ())),
                            preferred_element_type=jnp.float32)
        y_ref[...] = y * ws_ref[...]


_gmm = pl.pallas_call(
    _gmm_body,
    grid_spec=pltpu.PrefetchScalarGridSpec(
        num_scalar_prefetch=1,
        grid=(NT,),
        in_specs=[
            pl.BlockSpec((BT, D), lambda i, sp: (i, 0)),
            pl.BlockSpec((1, FF, D), lambda i, sp: (sp[i], 0, 0)),
            pl.BlockSpec((1, FF, D), lambda i, sp: (sp[i], 1, 0)),
            pl.BlockSpec((1, D, FF), lambda i, sp: (sp[i], 0, 0)),
            pl.BlockSpec((BT, 1), lambda i, sp: (i, 0)),
        ],
        out_specs=pl.BlockSpec((BT, D), lambda i, sp: (i, 0)),
    ),
    out_shape=jax.ShapeDtypeStruct((PP, D), jnp.float32),
)


def _shared_body(x_ref, w1_ref, w2_ref, o_ref):
    a = x_ref[...].astype(jnp.bfloat16)
    gu = lax.dot_general(a, w1_ref[...], (((1,), (1,)), ((), ())),
                         preferred_element_type=jnp.float32)
    g = gu[:, :SFF]
    u = gu[:, SFF:]
    act = (g * jax.nn.sigmoid(g) * u).astype(jnp.bfloat16)
    o_ref[...] = lax.dot_general(act, w2_ref[...], (((1,), (1,)), ((), ())),
                                 preferred_element_type=jnp.float32)


_shared = pl.pallas_call(
    _shared_body,
    grid=(T // BT,),
    in_specs=[
        pl.BlockSpec((BT, D), lambda i: (i, 0)),
        pl.BlockSpec((2 * SFF, D), lambda i: (0, 0)),
        pl.BlockSpec((D, SFF), lambda i: (0, 0)),
    ],
    out_specs=pl.BlockSpec((BT, D), lambda i: (i, 0)),
    out_shape=jax.ShapeDtypeStruct((T, D), jnp.float32),
)


def _combine_body(slot_hbm, y_hbm, sh_hbm, out_hbm,
                  s1_v, s2_v, y1_v, y2_v, o_v, sem1, sem2):
    wid = lax.axis_index("s") * NC + lax.axis_index("c")
    t0 = wid * TW
    pltpu.sync_copy(slot_hbm.at[pl.ds(t0, TW)], s1_v)
    pltpu.sync_copy(slot_hbm.at[pl.ds(T + t0, TW)], s2_v)

    def chunk_body(c, _):
        tc0 = c * 8
        cp1 = pltpu.async_copy(y_hbm.at[s1_v.at[pl.ds(tc0, 8)]], y1_v, sem1)
        cp2 = pltpu.async_copy(y_hbm.at[s2_v.at[pl.ds(tc0, 8)]], y2_v, sem2)
        pltpu.sync_copy(sh_hbm.at[pl.ds(t0 + tc0, 8)], o_v)
        cp1.wait()
        cp2.wait()
        for r in range(8):
            def add_body(j, _, r=r):
                sl = pl.ds(j * 16, 16)
                o_v[r, sl] = o_v[r, sl] + y1_v[r, sl] + y2_v[r, sl]
                return 0

            lax.fori_loop(0, D // 16, add_body, 0)
        pltpu.sync_copy(o_v, out_hbm.at[pl.ds(t0 + tc0, 8)])
        return 0

    lax.fori_loop(0, TW // 8, chunk_body, 0)


@functools.cache
def _sc_kernels():
    mesh = plsc.VectorSubcoreMesh(
        core_axis_name="c", subcore_axis_name="s",
        num_cores=NC, num_subcores=NS)
    sc_params = pltpu.CompilerParams(needs_layout_passes=False)
    dispatch = pl.kernel(
        _dispatch_body,
        compiler_params=sc_params,
        out_type=(
            jax.ShapeDtypeStruct((PP, D // 2), jnp.int32),
            jax.ShapeDtypeStruct((PP,), jnp.float32),
            jax.ShapeDtypeStruct((K * T,), jnp.int32),
        ),
        mesh=mesh,
        scratch_types=[
            pltpu.VMEM((4, T), jnp.int32),
            pltpu.VMEM((K, T), jnp.float32),
            pltpu.VMEM((16,), jnp.int32),
            pltpu.VMEM((PP,), jnp.int32),
            pltpu.VMEM((PP,), jnp.float32),
            pltpu.VMEM((K * T,), jnp.int32),
            pltpu.VMEM((24, D // 2), jnp.int32),
            pltpu.SemaphoreType.DMA,
        ],
    )
    combine = pl.kernel(
        _combine_body,
        compiler_params=sc_params,
        out_type=jax.ShapeDtypeStruct((T, D), jnp.float32),
        mesh=mesh,
        scratch_types=[
            pltpu.VMEM((TW,), jnp.int32),
            pltpu.VMEM((TW,), jnp.int32),
            pltpu.VMEM((8, D), jnp.float32),
            pltpu.VMEM((8, D), jnp.float32),
            pltpu.VMEM((8, D), jnp.float32),
            pltpu.SemaphoreType.DMA,
            pltpu.SemaphoreType.DMA,
        ],
    )
    return dispatch, combine


def _routing_metadata(cntf):
    cnt = cntf[0].astype(jnp.int32)
    pc = ((cnt + BT - 1) // BT) * BT
    cum = jnp.cumsum(pc)
    base = jnp.concatenate(
        [jnp.zeros((1,), jnp.int32), cum[:-1],
         jnp.zeros((16 - E,), jnp.int32)]).astype(jnp.int32)
    used = (cum[-1] // BT).astype(jnp.int32)
    tidx = jnp.arange(NT, dtype=jnp.int32) * BT
    te = jnp.minimum(
        jnp.sum((tidx[:, None] >= cum[None, :]).astype(jnp.int32), axis=1),
        E - 1)
    sp = jnp.concatenate([te, used[None]]).astype(jnp.int32)
    return base, sp


def kernel(hidden_states, gate_w, w1, w2, shared_w1, shared_w2):
    x = hidden_states.reshape(T, D)
    ints, ws, cntf = _router(x, gate_w)
    base, sp = _routing_metadata(cntf)

    x_i32 = lax.bitcast_convert_type(
        x.astype(jnp.bfloat16).reshape(T, D // 2, 2), jnp.int32)
    dispatch, combine = _sc_kernels()
    xs_i32, wsort, slots = dispatch(ints.T, ws.T, base, x_i32)
    xs_bf = lax.bitcast_convert_type(xs_i32, jnp.bfloat16).reshape(PP, D)

    w1_bf = w1.astype(jnp.bfloat16)
    w2_bf = w2.astype(jnp.bfloat16)
    y = _gmm(sp, xs_bf, w1_bf, w1_bf, w2_bf, wsort.reshape(PP, 1))

    sh = _shared(x, shared_w1.astype(jnp.bfloat16),
                 shared_w2.astype(jnp.bfloat16))
    out = combine(slots, y, sh)
    return out.reshape(1, T, D)


# speedup vs baseline: 1.4587x; 1.4587x over previous
import functools

import jax
import jax.numpy as jnp
from jax import lax
from jax.experimental import pallas as pl
from jax.experimental.pallas import tpu as pltpu
from jax.experimental.pallas import tpu_sc as plsc

E = 8
D = 2048
FF = 1408
SFF = 2816
T = 2048
K = 2
BT = 256
NT = 23
PP = NT * BT
NC = 2
NS = 16
NW = NC * NS
RW = PP // NW
TW = T // NW
TBR = 128


def _router_body(x_ref, g_ref, ints_ref, ws_ref, cnt_ref):
    i = pl.program_id(0)

    @pl.when(i == 0)
    def _():
        cnt_ref[...] = jnp.zeros_like(cnt_ref)

    x = x_ref[...]
    logits = lax.dot_general(
        x, g_ref[...], (((1,), (1,)), ((), ())),
        preferred_element_type=jnp.float32)

    tri = (lax.broadcasted_iota(jnp.int32, (E, E), 0)
           <= lax.broadcasted_iota(jnp.int32, (E, E), 1)).astype(jnp.float32)
    iota_e = lax.broadcasted_iota(jnp.int32, (TBR, E), 1).astype(jnp.float32)

    def pick(m):
        r = jnp.max(m, axis=1, keepdims=True)
        hit = (m == r).astype(jnp.float32)
        csum = lax.dot_general(hit, tri, (((1,), (0,)), ((), ())),
                               preferred_element_type=jnp.float32)
        first = hit * (csum == 1.0).astype(jnp.float32)
        e = jnp.sum(first * iota_e, axis=1, keepdims=True)
        return r, first, e

    r1, f1, e1 = pick(logits)
    r2, f2, e2 = pick(logits - f1 * 1e30)
    w_a = 1.0 / (1.0 + jnp.exp(r2 - r1))
    w_b = 1.0 / (1.0 + jnp.exp(r1 - r2))

    cnt = f1 + f2
    lower = (lax.broadcasted_iota(jnp.int32, (TBR, TBR), 1)
             < lax.broadcasted_iota(jnp.int32, (TBR, TBR), 0)).astype(jnp.float32)
    pos_in = lax.dot_general(lower, cnt, (((1,), (0,)), ((), ())),
                             preferred_element_type=jnp.float32)
    prev = cnt_ref[...]
    pos = pos_in + prev
    p1 = jnp.sum(f1 * pos, axis=1, keepdims=True)
    p2 = jnp.sum(f2 * pos, axis=1, keepdims=True)
    cnt_ref[...] = prev + jnp.sum(cnt, axis=0, keepdims=True)

    col4 = lax.broadcasted_iota(jnp.int32, (TBR, 4), 1)
    iv = jnp.where(col4 == 0, e1,
                   jnp.where(col4 == 1, e2, jnp.where(col4 == 2, p1, p2)))
    ints_ref[...] = iv.astype(jnp.int32)
    col2 = lax.broadcasted_iota(jnp.int32, (TBR, 2), 1)
    ws_ref[...] = jnp.where(col2 == 0, w_a, w_b)


_router = pl.pallas_call(
    _router_body,
    grid=(T // TBR,),
    in_specs=[
        pl.BlockSpec((TBR, D), lambda i: (i, 0)),
        pl.BlockSpec((E, D), lambda i: (0, 0)),
    ],
    out_specs=[
        pl.BlockSpec((TBR, 4), lambda i: (i, 0)),
        pl.BlockSpec((TBR, 2), lambda i: (i, 0)),
        pl.BlockSpec((1, E), lambda i: (0, 0)),
    ],
    out_shape=[
        jax.ShapeDtypeStruct((T, 4), jnp.int32),
        jax.ShapeDtypeStruct((T, 2), jnp.float32),
        jax.ShapeDtypeStruct((1, E), jnp.float32),
    ],
)


_CHUNKS = [(j * 16, 16) for j in range(11)] + [(176, 8)]


def _dispatch_body(ints_hbm, ws_hbm, base_hbm, x_hbm,
                   xs_hbm, wso_hbm, slot_hbm,
                   ints_v, ws_v, base_v, tok_v, wv, slots_v,
                   rows_a, rows_b, gsem_a, gsem_b, wsem_a, wsem_b):
    wid = lax.axis_index("s") * NC + lax.axis_index("c")
    pltpu.sync_copy(ints_hbm, ints_v)
    pltpu.sync_copy(ws_hbm, ws_v)
    pltpu.sync_copy(base_hbm, base_v)

    def zero_body(j, _):
        tok_v[pl.ds(j * 16, 16)] = jnp.zeros((16,), jnp.int32)
        wv[pl.ds(j * 16, 16)] = jnp.zeros((16,), jnp.float32)
        return 0

    lax.fori_loop(0, PP // 16, zero_body, 0)

    def scat_body(c, _):
        t0 = c * 16
        tok = lax.iota(jnp.int32, 16) + t0
        for k in range(K):
            e = ints_v[k, pl.ds(t0, 16)]
            p = ints_v[K + k, pl.ds(t0, 16)]
            w = ws_v[k, pl.ds(t0, 16)]
            slot = plsc.load_gather(base_v, [e]) + p
            plsc.store_scatter(tok_v, [slot], tok)
            plsc.store_scatter(wv, [slot], w)
            slots_v[pl.ds(k * T + t0, 16)] = slot
        return 0

    lax.fori_loop(0, T // 16, scat_body, 0)

    pltpu.sync_copy(wv.at[pl.ds(wid * RW, RW)], wso_hbm.at[pl.ds(wid * RW, RW)])

    @pl.when(wid == 0)
    def _():
        pltpu.sync_copy(slots_v, slot_hbm)

    bufs = (rows_a, rows_b)
    gsems = (gsem_a, gsem_b)
    wsems = (wsem_a, wsem_b)
    gcp = {}
    wcp = {}
    for j, (off, n) in enumerate(_CHUNKS):
        b = j % 2
        if j >= 2:
            wcp[j - 2].wait()
        idx = tok_v.at[pl.ds(wid * RW + off, n)]
        gcp[j] = pltpu.async_copy(x_hbm.at[idx], bufs[b].at[pl.ds(0, n)],
                                  gsems[b])
        if j >= 1:
            po, pn = _CHUNKS[j - 1]
            gcp[j - 1].wait()
            wcp[j - 1] = pltpu.async_copy(
                bufs[1 - b].at[pl.ds(0, pn)],
                xs_hbm.at[pl.ds(wid * RW + po, pn)], wsems[1 - b])
    last = len(_CHUNKS) - 1
    lo, ln = _CHUNKS[last]
    gcp[last].wait()
    wcp[last] = pltpu.async_copy(bufs[last % 2].at[pl.ds(0, ln)],
                                 xs_hbm.at[pl.ds(wid * RW + lo, ln)],
                                 wsems[last % 2])
    wcp[last - 1].wait()
    wcp[last].wait()


def _gmm_body(sp_ref, xs_ref, w1g_ref, w1u_ref, w2_ref, ws_ref, y_ref):
    i = pl.program_id(0)

    @pl.when(i < sp_ref[NT])
    def _():
        a = xs_ref[...].astype(jnp.bfloat16)
        g = lax.dot_general(a, w1g_ref[0], (((1,), (1,)), ((), ())),
                            preferred_element_type=jnp.float32)
        u = lax.dot_general(a, w1u_ref[0], (((1,), (1,)), ((), ())),
                            preferred_element_type=jnp.float32)
        act = (g * jax.nn.sigmoid(g) * u).astype(jnp.bfloat16)
        y = lax.dot_general(act, w2_ref[0], (((1,), (1,)), ((), ())),
                            preferred_element_type=jnp.float32)
        y_ref[...] = y * ws_ref[...]


_gmm = pl.pallas_call(
    _gmm_body,
    grid_spec=pltpu.PrefetchScalarGridSpec(
        num_scalar_prefetch=1,
        grid=(NT,),
        in_specs=[
            pl.BlockSpec((BT, D), lambda i, sp: (i, 0)),
            pl.BlockSpec((1, FF, D), lambda i, sp: (sp[i], 0, 0)),
            pl.BlockSpec((1, FF, D), lambda i, sp: (sp[i], 1, 0)),
            pl.BlockSpec((1, D, FF), lambda i, sp: (sp[i], 0, 0)),
            pl.BlockSpec((BT, 1), lambda i, sp: (i, 0)),
        ],
        out_specs=pl.BlockSpec((BT, D), lambda i, sp: (i, 0)),
    ),
    out_shape=jax.ShapeDtypeStruct((PP, D), jnp.float32),
)


def _shared_body(x_ref, w1_ref, w2_ref, o_ref):
    a = x_ref[...].astype(jnp.bfloat16)
    gu = lax.dot_general(a, w1_ref[...], (((1,), (1,)), ((), ())),
                         preferred_element_type=jnp.float32)
    g = gu[:, :SFF]
    u = gu[:, SFF:]
    act = (g * jax.nn.sigmoid(g) * u).astype(jnp.bfloat16)
    o_ref[...] = lax.dot_general(act, w2_ref[...], (((1,), (1,)), ((), ())),
                                 preferred_element_type=jnp.float32)


_shared = pl.pallas_call(
    _shared_body,
    grid=(T // BT,),
    in_specs=[
        pl.BlockSpec((BT, D), lambda i: (i, 0)),
        pl.BlockSpec((2 * SFF, D), lambda i: (0, 0)),
        pl.BlockSpec((D, SFF), lambda i: (0, 0)),
    ],
    out_specs=pl.BlockSpec((BT, D), lambda i: (i, 0)),
    out_shape=jax.ShapeDtypeStruct((T, D), jnp.float32),
)


def _combine_body(slot_hbm, y_hbm, sh_hbm, out_hbm,
                  s1_v, s2_v, y1_v, y2_v, o_v, sem1, sem2):
    wid = lax.axis_index("s") * NC + lax.axis_index("c")
    t0 = wid * TW
    pltpu.sync_copy(slot_hbm.at[pl.ds(t0, TW)], s1_v)
    pltpu.sync_copy(slot_hbm.at[pl.ds(T + t0, TW)], s2_v)

    def chunk_body(c, _):
        tc0 = c * 8
        cp1 = pltpu.async_copy(y_hbm.at[s1_v.at[pl.ds(tc0, 8)]], y1_v, sem1)
        cp2 = pltpu.async_copy(y_hbm.at[s2_v.at[pl.ds(tc0, 8)]], y2_v, sem2)
        pltpu.sync_copy(sh_hbm.at[pl.ds(t0 + tc0, 8)], o_v)
        cp1.wait()
        cp2.wait()
        for r in range(8):
            def add_body(j, _, r=r):
                sl = pl.ds(j * 16, 16)
                o_v[r, sl] = o_v[r, sl] + y1_v[r, sl] + y2_v[r, sl]
                return 0

            lax.fori_loop(0, D // 16, add_body, 0)
        pltpu.sync_copy(o_v, out_hbm.at[pl.ds(t0 + tc0, 8)])
        return 0

    lax.fori_loop(0, TW // 8, chunk_body, 0)


@functools.cache
def _sc_kernels():
    mesh = plsc.VectorSubcoreMesh(
        core_axis_name="c", subcore_axis_name="s",
        num_cores=NC, num_subcores=NS)
    sc_params = pltpu.CompilerParams(needs_layout_passes=False)
    dispatch = pl.kernel(
        _dispatch_body,
        compiler_params=sc_params,
        out_type=(
            jax.ShapeDtypeStruct((PP, D), jnp.float32),
            jax.ShapeDtypeStruct((PP,), jnp.float32),
            jax.ShapeDtypeStruct((K * T,), jnp.int32),
        ),
        mesh=mesh,
        scratch_types=[
            pltpu.VMEM((4, T), jnp.int32),
            pltpu.VMEM((K, T), jnp.float32),
            pltpu.VMEM((16,), jnp.int32),
            pltpu.VMEM((PP,), jnp.int32),
            pltpu.VMEM((PP,), jnp.float32),
            pltpu.VMEM((K * T,), jnp.int32),
            pltpu.VMEM((16, D), jnp.float32),
            pltpu.VMEM((16, D), jnp.float32),
            pltpu.SemaphoreType.DMA,
            pltpu.SemaphoreType.DMA,
            pltpu.SemaphoreType.DMA,
            pltpu.SemaphoreType.DMA,
        ],
    )
    combine = pl.kernel(
        _combine_body,
        compiler_params=sc_params,
        out_type=jax.ShapeDtypeStruct((T, D), jnp.float32),
        mesh=mesh,
        scratch_types=[
            pltpu.VMEM((TW,), jnp.int32),
            pltpu.VMEM((TW,), jnp.int32),
            pltpu.VMEM((8, D), jnp.float32),
            pltpu.VMEM((8, D), jnp.float32),
            pltpu.VMEM((8, D), jnp.float32),
            pltpu.SemaphoreType.DMA,
            pltpu.SemaphoreType.DMA,
        ],
    )
    return dispatch, combine


def _cast_body(a_ref, o_ref):
    o_ref[...] = a_ref[...].astype(jnp.bfloat16)


def _make_cast(n, r, c):
    return pl.pallas_call(
        _cast_body,
        grid=(n,),
        in_specs=[pl.BlockSpec((1, r, c), lambda i: (i, 0, 0))],
        out_specs=pl.BlockSpec((1, r, c), lambda i: (i, 0, 0)),
        out_shape=jax.ShapeDtypeStruct((n, r, c), jnp.bfloat16),
    )


_cast_w1 = _make_cast(16, FF, D)
_cast_w2 = _make_cast(16, D // 2, FF)
_cast_sw1 = _make_cast(8, 2 * SFF // 8, D)
_cast_sw2 = _make_cast(8, D // 8, SFF)


def _routing_metadata(cntf):
    cnt = cntf[0].astype(jnp.int32)
    pc = ((cnt + BT - 1) // BT) * BT
    cum = jnp.cumsum(pc)
    base = jnp.concatenate(
        [jnp.zeros((1,), jnp.int32), cum[:-1],
         jnp.zeros((16 - E,), jnp.int32)]).astype(jnp.int32)
    used = (cum[-1] // BT).astype(jnp.int32)
    tidx = jnp.arange(NT, dtype=jnp.int32) * BT
    te = jnp.minimum(
        jnp.sum((tidx[:, None] >= cum[None, :]).astype(jnp.int32), axis=1),
        E - 1)
    sp = jnp.concatenate([te, used[None]]).astype(jnp.int32)
    return base, sp


def kernel(hidden_states, gate_w, w1, w2, shared_w1, shared_w2):
    x = hidden_states.reshape(T, D)
    ints, ws, cntf = _router(x, gate_w)
    base, sp = _routing_metadata(cntf)

    sw1_bf = _cast_sw1(shared_w1.reshape(8, 2 * SFF // 8, D)).reshape(2 * SFF, D)
    sw2_bf = _cast_sw2(shared_w2.reshape(8, D // 8, SFF)).reshape(D, SFF)
    sh = _shared(x, sw1_bf, sw2_bf)

    dispatch, combine = _sc_kernels()
    xs, wsort, slots = dispatch(ints.T, ws.T, base, x)

    w1_bf = _cast_w1(w1.reshape(16, FF, D)).reshape(E, 2 * FF, D)
    w2_bf = _cast_w2(w2.reshape(16, D // 2, FF)).reshape(E, D, FF)
    y = _gmm(sp, xs, w1_bf, w1_bf, w2_bf, wsort.reshape(PP, 1))

    out = combine(slots, y, sh)
    return out.reshape(1, T, D)


# speedup vs baseline: 1.7550x; 1.2031x over previous
import functools

import jax
import jax.numpy as jnp
from jax import lax
from jax.experimental import pallas as pl
from jax.experimental.pallas import tpu as pltpu
from jax.experimental.pallas import tpu_sc as plsc

E = 8
D = 2048
FF = 1408
SFF = 2816
T = 2048
K = 2
BT = 256
NT = 23
PP = NT * BT
NC = 2
NS = 16
NW = NC * NS
RW = PP // NW
TW = T // NW
TBR = 128


def _router_body(x_ref, g_ref, ints_ref, ws_ref, cnt_ref):
    i = pl.program_id(0)

    @pl.when(i == 0)
    def _():
        cnt_ref[...] = jnp.zeros_like(cnt_ref)

    x = x_ref[...]
    logits = lax.dot_general(
        x, g_ref[...], (((1,), (1,)), ((), ())),
        preferred_element_type=jnp.float32)

    tri = (lax.broadcasted_iota(jnp.int32, (E, E), 0)
           <= lax.broadcasted_iota(jnp.int32, (E, E), 1)).astype(jnp.float32)
    iota_e = lax.broadcasted_iota(jnp.int32, (TBR, E), 1).astype(jnp.float32)

    def pick(m):
        r = jnp.max(m, axis=1, keepdims=True)
        hit = (m == r).astype(jnp.float32)
        csum = lax.dot_general(hit, tri, (((1,), (0,)), ((), ())),
                               preferred_element_type=jnp.float32)
        first = hit * (csum == 1.0).astype(jnp.float32)
        e = jnp.sum(first * iota_e, axis=1, keepdims=True)
        return r, first, e

    r1, f1, e1 = pick(logits)
    r2, f2, e2 = pick(logits - f1 * 1e30)
    w_a = 1.0 / (1.0 + jnp.exp(r2 - r1))
    w_b = 1.0 / (1.0 + jnp.exp(r1 - r2))

    cnt = f1 + f2
    lower = (lax.broadcasted_iota(jnp.int32, (TBR, TBR), 1)
             < lax.broadcasted_iota(jnp.int32, (TBR, TBR), 0)).astype(jnp.float32)
    pos_in = lax.dot_general(lower, cnt, (((1,), (0,)), ((), ())),
                             preferred_element_type=jnp.float32)
    prev = cnt_ref[...]
    pos = pos_in + prev
    p1 = jnp.sum(f1 * pos, axis=1, keepdims=True)
    p2 = jnp.sum(f2 * pos, axis=1, keepdims=True)
    cnt_ref[...] = prev + jnp.sum(cnt, axis=0, keepdims=True)

    col4 = lax.broadcasted_iota(jnp.int32, (TBR, 4), 1)
    iv = jnp.where(col4 == 0, e1,
                   jnp.where(col4 == 1, e2, jnp.where(col4 == 2, p1, p2)))
    ints_ref[...] = iv.astype(jnp.int32)
    col2 = lax.broadcasted_iota(jnp.int32, (TBR, 2), 1)
    ws_ref[...] = jnp.where(col2 == 0, w_a, w_b)


_router = pl.pallas_call(
    _router_body,
    grid=(T // TBR,),
    in_specs=[
        pl.BlockSpec((TBR, D), lambda i: (i, 0)),
        pl.BlockSpec((E, D), lambda i: (0, 0)),
    ],
    out_specs=[
        pl.BlockSpec((TBR, 4), lambda i: (i, 0)),
        pl.BlockSpec((TBR, 2), lambda i: (i, 0)),
        pl.BlockSpec((1, E), lambda i: (0, 0)),
    ],
    out_shape=[
        jax.ShapeDtypeStruct((T, 4), jnp.int32),
        jax.ShapeDtypeStruct((T, 2), jnp.float32),
        jax.ShapeDtypeStruct((1, E), jnp.float32),
    ],
)


_CHUNKS = [(j * 16, 16) for j in range(11)] + [(176, 8)]


def _dispatch_body(ints_hbm, ws_hbm, base_hbm, x_hbm,
                   xs_hbm, wso_hbm, slot_hbm,
                   ints_v, ws_v, base_v, tok_v, wv, slots_v,
                   rows_a, rows_b, gsem_a, gsem_b, wsem_a, wsem_b):
    wid = lax.axis_index("s") * NC + lax.axis_index("c")
    pltpu.sync_copy(ints_hbm, ints_v)
    pltpu.sync_copy(ws_hbm, ws_v)
    pltpu.sync_copy(base_hbm, base_v)

    def zero_body(j, _):
        tok_v[pl.ds(j * 16, 16)] = jnp.zeros((16,), jnp.int32)
        wv[pl.ds(j * 16, 16)] = jnp.zeros((16,), jnp.float32)
        return 0

    lax.fori_loop(0, PP // 16, zero_body, 0)

    def scat_body(c, _):
        t0 = c * 16
        tok = lax.iota(jnp.int32, 16) + t0
        for k in range(K):
            e = ints_v[k, pl.ds(t0, 16)]
            p = ints_v[K + k, pl.ds(t0, 16)]
            w = ws_v[k, pl.ds(t0, 16)]
            slot = plsc.load_gather(base_v, [e]) + p
            plsc.store_scatter(tok_v, [slot], tok)
            plsc.store_scatter(wv, [slot], w)
            slots_v[pl.ds(k * T + t0, 16)] = slot
        return 0

    lax.fori_loop(0, T // 16, scat_body, 0)

    pltpu.sync_copy(wv.at[pl.ds(wid * RW, RW)], wso_hbm.at[pl.ds(wid * RW, RW)])

    @pl.when(wid == 0)
    def _():
        pltpu.sync_copy(slots_v, slot_hbm)

    bufs = (rows_a, rows_b)
    gsems = (gsem_a, gsem_b)
    wsems = (wsem_a, wsem_b)
    for j, (off, n) in enumerate(_CHUNKS):
        b = j % 2
        woff = min(off, RW - 16)
        live = jnp.max(jnp.abs(wv[pl.ds(wid * RW + woff, 16)])) > 0.0

        @pl.when(live)
        def _(b=b, off=off, n=n):
            idx = tok_v.at[pl.ds(wid * RW + off, n)]
            pltpu.async_copy(x_hbm.at[idx], bufs[b].at[pl.ds(0, n)],
                             gsems[b]).wait()
            pltpu.async_copy(bufs[b].at[pl.ds(0, n)],
                             xs_hbm.at[pl.ds(wid * RW + off, n)],
                             wsems[b]).wait()


def _gmm_body(sp_ref, xs_ref, w1g_ref, w1u_ref, w2_ref, ws_ref, y_ref):
    i = pl.program_id(0)

    @pl.when(i < sp_ref[NT])
    def _():
        a = xs_ref[...].astype(jnp.bfloat16)
        g = lax.dot_general(a, w1g_ref[0], (((1,), (1,)), ((), ())),
                            preferred_element_type=jnp.float32)
        u = lax.dot_general(a, w1u_ref[0], (((1,), (1,)), ((), ())),
                            preferred_element_type=jnp.float32)
        act = (g * jax.nn.sigmoid(g) * u).astype(jnp.bfloat16)
        y = lax.dot_general(act, w2_ref[0], (((1,), (1,)), ((), ())),
                            preferred_element_type=jnp.float32)
        y_ref[...] = y * ws_ref[...]


_gmm = pl.pallas_call(
    _gmm_body,
    grid_spec=pltpu.PrefetchScalarGridSpec(
        num_scalar_prefetch=1,
        grid=(NT,),
        in_specs=[
            pl.BlockSpec((BT, D), lambda i, sp: (i, 0)),
            pl.BlockSpec((1, FF, D), lambda i, sp: (sp[i], 0, 0)),
            pl.BlockSpec((1, FF, D), lambda i, sp: (sp[i], 1, 0)),
            pl.BlockSpec((1, D, FF), lambda i, sp: (sp[i], 0, 0)),
            pl.BlockSpec((BT, 1), lambda i, sp: (i, 0)),
        ],
        out_specs=pl.BlockSpec((BT, D), lambda i, sp: (i, 0)),
    ),
    out_shape=jax.ShapeDtypeStruct((PP, D), jnp.float32),
)


def _shared_body(x_ref, w1_ref, w2_ref, o_ref):
    a = x_ref[...].astype(jnp.bfloat16)
    gu = lax.dot_general(a, w1_ref[...], (((1,), (1,)), ((), ())),
                         preferred_element_type=jnp.float32)
    g = gu[:, :SFF]
    u = gu[:, SFF:]
    act = (g * jax.nn.sigmoid(g) * u).astype(jnp.bfloat16)
    o_ref[...] = lax.dot_general(act, w2_ref[...], (((1,), (1,)), ((), ())),
                                 preferred_element_type=jnp.float32)


_shared = pl.pallas_call(
    _shared_body,
    grid=(T // BT,),
    in_specs=[
        pl.BlockSpec((BT, D), lambda i: (i, 0)),
        pl.BlockSpec((2 * SFF, D), lambda i: (0, 0)),
        pl.BlockSpec((D, SFF), lambda i: (0, 0)),
    ],
    out_specs=pl.BlockSpec((BT, D), lambda i: (i, 0)),
    out_shape=jax.ShapeDtypeStruct((T, D), jnp.float32),
)


def _combine_body(slot_hbm, y_hbm, sh_hbm, out_hbm,
                  s1_v, s2_v, y1_v, y2_v, o_v, sem1, sem2):
    wid = lax.axis_index("s") * NC + lax.axis_index("c")
    t0 = wid * TW
    pltpu.sync_copy(slot_hbm.at[pl.ds(t0, TW)], s1_v)
    pltpu.sync_copy(slot_hbm.at[pl.ds(T + t0, TW)], s2_v)

    def chunk_body(c, _):
        tc0 = c * 8
        cp1 = pltpu.async_copy(y_hbm.at[s1_v.at[pl.ds(tc0, 8)]], y1_v, sem1)
        cp2 = pltpu.async_copy(y_hbm.at[s2_v.at[pl.ds(tc0, 8)]], y2_v, sem2)
        pltpu.sync_copy(sh_hbm.at[pl.ds(t0 + tc0, 8)], o_v)
        cp1.wait()
        cp2.wait()
        for r in range(8):
            def add_body(j, _, r=r):
                sl = pl.ds(j * 16, 16)
                o_v[r, sl] = o_v[r, sl] + y1_v[r, sl] + y2_v[r, sl]
                return 0

            lax.fori_loop(0, D // 16, add_body, 0)
        pltpu.sync_copy(o_v, out_hbm.at[pl.ds(t0 + tc0, 8)])
        return 0

    lax.fori_loop(0, TW // 8, chunk_body, 0)


@functools.cache
def _sc_kernels():
    mesh = plsc.VectorSubcoreMesh(
        core_axis_name="c", subcore_axis_name="s",
        num_cores=NC, num_subcores=NS)
    sc_params = pltpu.CompilerParams(needs_layout_passes=False)
    dispatch = pl.kernel(
        _dispatch_body,
        compiler_params=sc_params,
        out_type=(
            jax.ShapeDtypeStruct((PP, D), jnp.float32),
            jax.ShapeDtypeStruct((PP,), jnp.float32),
            jax.ShapeDtypeStruct((K * T,), jnp.int32),
        ),
        mesh=mesh,
        scratch_types=[
            pltpu.VMEM((4, T), jnp.int32),
            pltpu.VMEM((K, T), jnp.float32),
            pltpu.VMEM((16,), jnp.int32),
            pltpu.VMEM((PP,), jnp.int32),
            pltpu.VMEM((PP,), jnp.float32),
            pltpu.VMEM((K * T,), jnp.int32),
            pltpu.VMEM((16, D), jnp.float32),
            pltpu.VMEM((16, D), jnp.float32),
            pltpu.SemaphoreType.DMA,
            pltpu.SemaphoreType.DMA,
            pltpu.SemaphoreType.DMA,
            pltpu.SemaphoreType.DMA,
        ],
    )
    combine = pl.kernel(
        _combine_body,
        compiler_params=sc_params,
        out_type=jax.ShapeDtypeStruct((T, D), jnp.float32),
        mesh=mesh,
        scratch_types=[
            pltpu.VMEM((TW,), jnp.int32),
            pltpu.VMEM((TW,), jnp.int32),
            pltpu.VMEM((8, D), jnp.float32),
            pltpu.VMEM((8, D), jnp.float32),
            pltpu.VMEM((8, D), jnp.float32),
            pltpu.SemaphoreType.DMA,
            pltpu.SemaphoreType.DMA,
        ],
    )
    return dispatch, combine


def _cast_body(a_ref, o_ref):
    o_ref[...] = a_ref[...].astype(jnp.bfloat16)


def _make_cast(n, r, c):
    return pl.pallas_call(
        _cast_body,
        grid=(n,),
        in_specs=[pl.BlockSpec((1, r, c), lambda i: (i, 0, 0))],
        out_specs=pl.BlockSpec((1, r, c), lambda i: (i, 0, 0)),
        out_shape=jax.ShapeDtypeStruct((n, r, c), jnp.bfloat16),
    )


_cast_w1 = _make_cast(32, FF // 2, D)
_cast_w2 = _make_cast(16, D // 2, FF)
_cast_sw1 = _make_cast(8, 2 * SFF // 8, D)
_cast_sw2 = _make_cast(8, D // 8, SFF)


def _routing_metadata(cntf):
    cnt = cntf[0].astype(jnp.int32)
    pc = ((cnt + BT - 1) // BT) * BT
    cum = jnp.cumsum(pc)
    base = jnp.concatenate(
        [jnp.zeros((1,), jnp.int32), cum[:-1],
         jnp.zeros((16 - E,), jnp.int32)]).astype(jnp.int32)
    used = (cum[-1] // BT).astype(jnp.int32)
    tidx = jnp.arange(NT, dtype=jnp.int32) * BT
    te = jnp.minimum(
        jnp.sum((tidx[:, None] >= cum[None, :]).astype(jnp.int32), axis=1),
        E - 1)
    sp = jnp.concatenate([te, used[None]]).astype(jnp.int32)
    return base, sp


def kernel(hidden_states, gate_w, w1, w2, shared_w1, shared_w2):
    x = hidden_states.reshape(T, D)
    ints, ws, cntf = _router(x, gate_w)
    base, sp = _routing_metadata(cntf)

    sw1_bf = _cast_sw1(shared_w1.reshape(8, 2 * SFF // 8, D)).reshape(2 * SFF, D)
    sw2_bf = _cast_sw2(shared_w2.reshape(8, D // 8, SFF)).reshape(D, SFF)
    sh = _shared(x, sw1_bf, sw2_bf)

    dispatch, combine = _sc_kernels()
    xs, wsort, slots = dispatch(ints.T, ws.T, base, x)

    w1_bf = _cast_w1(w1.reshape(32, FF // 2, D)).reshape(E, 2 * FF, D)
    w2_bf = _cast_w2(w2.reshape(16, D // 2, FF)).reshape(E, D, FF)
    y = _gmm(sp, xs, w1_bf, w1_bf, w2_bf, wsort.reshape(PP, 1))

    out = combine(slots, y, sh)
    return out.reshape(1, T, D)


# speedup vs baseline: 1.8633x; 1.0617x over previous
import functools

import jax
import jax.numpy as jnp
from jax import lax
from jax.experimental import pallas as pl
from jax.experimental.pallas import tpu as pltpu
from jax.experimental.pallas import tpu_sc as plsc

E = 8
D = 2048
FF = 1408
SFF = 2816
T = 2048
K = 2
BT = 256
NT = 23
PP = NT * BT
NC = 2
NS = 16
NW = NC * NS
RW = PP // NW
TW = T // NW
TBR = 128


def _router_body(x_ref, g_ref, ints_ref, ws_ref, cnt_ref):
    i = pl.program_id(0)

    @pl.when(i == 0)
    def _():
        cnt_ref[...] = jnp.zeros_like(cnt_ref)

    x = x_ref[...]
    logits = lax.dot_general(
        x, g_ref[...], (((1,), (1,)), ((), ())),
        preferred_element_type=jnp.float32)

    tri = (lax.broadcasted_iota(jnp.int32, (E, E), 0)
           <= lax.broadcasted_iota(jnp.int32, (E, E), 1)).astype(jnp.float32)
    iota_e = lax.broadcasted_iota(jnp.int32, (TBR, E), 1).astype(jnp.float32)

    def pick(m):
        r = jnp.max(m, axis=1, keepdims=True)
        hit = (m == r).astype(jnp.float32)
        csum = lax.dot_general(hit, tri, (((1,), (0,)), ((), ())),
                               preferred_element_type=jnp.float32)
        first = hit * (csum == 1.0).astype(jnp.float32)
        e = jnp.sum(first * iota_e, axis=1, keepdims=True)
        return r, first, e

    r1, f1, e1 = pick(logits)
    r2, f2, e2 = pick(logits - f1 * 1e30)
    w_a = 1.0 / (1.0 + jnp.exp(r2 - r1))
    w_b = 1.0 / (1.0 + jnp.exp(r1 - r2))

    cnt = f1 + f2
    lower = (lax.broadcasted_iota(jnp.int32, (TBR, TBR), 1)
             < lax.broadcasted_iota(jnp.int32, (TBR, TBR), 0)).astype(jnp.float32)
    pos_in = lax.dot_general(lower, cnt, (((1,), (0,)), ((), ())),
                             preferred_element_type=jnp.float32)
    prev = cnt_ref[...]
    pos = pos_in + prev
    p1 = jnp.sum(f1 * pos, axis=1, keepdims=True)
    p2 = jnp.sum(f2 * pos, axis=1, keepdims=True)
    cnt_ref[...] = prev + jnp.sum(cnt, axis=0, keepdims=True)

    col4 = lax.broadcasted_iota(jnp.int32, (TBR, 4), 1)
    iv = jnp.where(col4 == 0, e1,
                   jnp.where(col4 == 1, e2, jnp.where(col4 == 2, p1, p2)))
    ints_ref[...] = iv.astype(jnp.int32)
    col2 = lax.broadcasted_iota(jnp.int32, (TBR, 2), 1)
    ws_ref[...] = jnp.where(col2 == 0, w_a, w_b)


_router = pl.pallas_call(
    _router_body,
    grid=(T // TBR,),
    in_specs=[
        pl.BlockSpec((TBR, D), lambda i: (i, 0)),
        pl.BlockSpec((E, D), lambda i: (0, 0)),
    ],
    out_specs=[
        pl.BlockSpec((TBR, 4), lambda i: (i, 0)),
        pl.BlockSpec((TBR, 2), lambda i: (i, 0)),
        pl.BlockSpec((1, E), lambda i: (0, 0)),
    ],
    out_shape=[
        jax.ShapeDtypeStruct((T, 4), jnp.int32),
        jax.ShapeDtypeStruct((T, 2), jnp.float32),
        jax.ShapeDtypeStruct((1, E), jnp.float32),
    ],
)


_CHUNKS = [(j * 16, 16) for j in range(11)] + [(176, 8)]


def _dispatch_body(ints_hbm, ws_hbm, base_hbm, x_hbm,
                   xs_hbm, wso_hbm, slot_hbm,
                   ints_v, ws_v, base_v, tok_v, wv, slots_v,
                   rows_a, rows_b, gsem_a, gsem_b, wsem_a, wsem_b):
    wid = lax.axis_index("s") * NC + lax.axis_index("c")
    pltpu.sync_copy(ints_hbm, ints_v)
    pltpu.sync_copy(ws_hbm, ws_v)
    pltpu.sync_copy(base_hbm, base_v)

    def zero_body(j, _):
        tok_v[pl.ds(j * 16, 16)] = jnp.zeros((16,), jnp.int32)
        wv[pl.ds(j * 16, 16)] = jnp.zeros((16,), jnp.float32)
        return 0

    lax.fori_loop(0, PP // 16, zero_body, 0)

    def scat_body(c, _):
        t0 = c * 16
        tok = lax.iota(jnp.int32, 16) + t0
        for k in range(K):
            e = ints_v[k, pl.ds(t0, 16)]
            p = ints_v[K + k, pl.ds(t0, 16)]
            w = ws_v[k, pl.ds(t0, 16)]
            slot = plsc.load_gather(base_v, [e]) + p
            plsc.store_scatter(tok_v, [slot], tok)
            plsc.store_scatter(wv, [slot], w)
            slots_v[pl.ds(k * T + t0, 16)] = slot
        return 0

    lax.fori_loop(0, T // 16, scat_body, 0)

    pltpu.sync_copy(wv.at[pl.ds(wid * RW, RW)], wso_hbm.at[pl.ds(wid * RW, RW)])

    @pl.when(wid == 0)
    def _():
        pltpu.sync_copy(slots_v, slot_hbm)

    bufs = (rows_a, rows_b)
    gsems = (gsem_a, gsem_b)
    wsems = (wsem_a, wsem_b)
    for j, (off, n) in enumerate(_CHUNKS):
        b = j % 2
        woff = min(off, RW - 16)
        live = jnp.max(jnp.abs(wv[pl.ds(wid * RW + woff, 16)])) > 0.0

        @pl.when(live)
        def _(b=b, off=off, n=n):
            idx = tok_v.at[pl.ds(wid * RW + off, n)]
            pltpu.async_copy(x_hbm.at[idx], bufs[b].at[pl.ds(0, n)],
                             gsems[b]).wait()
            pltpu.async_copy(bufs[b].at[pl.ds(0, n)],
                             xs_hbm.at[pl.ds(wid * RW + off, n)],
                             wsems[b]).wait()


def _gmm_body(sp_ref, xs_ref, w1g_ref, w1u_ref, w2_ref, ws_ref, y_ref):
    i = pl.program_id(0)

    @pl.when(i < sp_ref[NT])
    def _():
        a = xs_ref[...].astype(jnp.bfloat16)
        g = lax.dot_general(a, w1g_ref[0], (((1,), (1,)), ((), ())),
                            preferred_element_type=jnp.float32)
        u = lax.dot_general(a, w1u_ref[0], (((1,), (1,)), ((), ())),
                            preferred_element_type=jnp.float32)
        act = (g * jax.nn.sigmoid(g) * u).astype(jnp.bfloat16)
        y = lax.dot_general(act, w2_ref[0], (((1,), (1,)), ((), ())),
                            preferred_element_type=jnp.float32)
        y_ref[...] = y * ws_ref[...]


_gmm = pl.pallas_call(
    _gmm_body,
    grid_spec=pltpu.PrefetchScalarGridSpec(
        num_scalar_prefetch=1,
        grid=(NT,),
        in_specs=[
            pl.BlockSpec((BT, D), lambda i, sp: (i, 0)),
            pl.BlockSpec((1, FF, D), lambda i, sp: (sp[i], 0, 0)),
            pl.BlockSpec((1, FF, D), lambda i, sp: (sp[i], 1, 0)),
            pl.BlockSpec((1, D, FF), lambda i, sp: (sp[i], 0, 0)),
            pl.BlockSpec((BT, 1), lambda i, sp: (i, 0)),
        ],
        out_specs=pl.BlockSpec((BT, D), lambda i, sp: (i, 0)),
    ),
    out_shape=jax.ShapeDtypeStruct((PP, D), jnp.float32),
)


def _shared_body(x_ref, w1_ref, w2_ref, o_ref):
    a = x_ref[...].astype(jnp.bfloat16)
    gu = lax.dot_general(a, w1_ref[...], (((1,), (1,)), ((), ())),
                         preferred_element_type=jnp.float32)
    g = gu[:, :SFF]
    u = gu[:, SFF:]
    act = (g * jax.nn.sigmoid(g) * u).astype(jnp.bfloat16)
    o_ref[...] = lax.dot_general(act, w2_ref[...], (((1,), (1,)), ((), ())),
                                 preferred_element_type=jnp.float32)


_shared = pl.pallas_call(
    _shared_body,
    grid=(T // BT,),
    in_specs=[
        pl.BlockSpec((BT, D), lambda i: (i, 0)),
        pl.BlockSpec((2 * SFF, D), lambda i: (0, 0)),
        pl.BlockSpec((D, SFF), lambda i: (0, 0)),
    ],
    out_specs=pl.BlockSpec((BT, D), lambda i: (i, 0)),
    out_shape=jax.ShapeDtypeStruct((T, D), jnp.float32),
)


def _combine_body(slot_hbm, y_hbm, sh_hbm, out_hbm,
                  s1_v, s2_v, y1a, y1b, y2a, y2b, sha, shb,
                  g1a, g1b, g2a, g2b, gha, ghb, wsa, wsb):
    wid = lax.axis_index("s") * NC + lax.axis_index("c")
    t0 = wid * TW
    pltpu.sync_copy(slot_hbm.at[pl.ds(t0, TW)], s1_v)
    pltpu.sync_copy(slot_hbm.at[pl.ds(T + t0, TW)], s2_v)

    y1 = (y1a, y1b)
    y2 = (y2a, y2b)
    sh = (sha, shb)
    g1 = (g1a, g1b)
    g2 = (g2a, g2b)
    gh = (gha, ghb)
    wsems = (wsa, wsb)
    nch = TW // 8

    def issue(c):
        b = c % 2
        tc0 = c * 8
        return (pltpu.async_copy(y_hbm.at[s1_v.at[pl.ds(tc0, 8)]], y1[b], g1[b]),
                pltpu.async_copy(y_hbm.at[s2_v.at[pl.ds(tc0, 8)]], y2[b], g2[b]),
                pltpu.async_copy(sh_hbm.at[pl.ds(t0 + tc0, 8)], sh[b], gh[b]))

    cps = {0: issue(0)}
    wcp = {}
    for c in range(nch):
        b = c % 2
        if c + 1 < nch:
            if c >= 1:
                wcp[c - 1].wait()
            cps[c + 1] = issue(c + 1)
        for cp in cps.pop(c):
            cp.wait()
        for r in range(8):
            def add_body(j, _, r=r, b=b):
                s0 = pl.ds(j * 32, 16)
                s1 = pl.ds(j * 32 + 16, 16)
                sh[b][r, s0] = sh[b][r, s0] + y1[b][r, s0] + y2[b][r, s0]
                sh[b][r, s1] = sh[b][r, s1] + y1[b][r, s1] + y2[b][r, s1]
                return 0

            lax.fori_loop(0, D // 32, add_body, 0)
        wcp[c] = pltpu.async_copy(sh[b], out_hbm.at[pl.ds(t0 + c * 8, 8)],
                                  wsems[b])
    wcp[nch - 2].wait()
    wcp[nch - 1].wait()


@functools.cache
def _sc_kernels():
    mesh = plsc.VectorSubcoreMesh(
        core_axis_name="c", subcore_axis_name="s",
        num_cores=NC, num_subcores=NS)
    sc_params = pltpu.CompilerParams(needs_layout_passes=False)
    dispatch = pl.kernel(
        _dispatch_body,
        compiler_params=sc_params,
        out_type=(
            jax.ShapeDtypeStruct((PP, D), jnp.float32),
            jax.ShapeDtypeStruct((PP,), jnp.float32),
            jax.ShapeDtypeStruct((K * T,), jnp.int32),
        ),
        mesh=mesh,
        scratch_types=[
            pltpu.VMEM((4, T), jnp.int32),
            pltpu.VMEM((K, T), jnp.float32),
            pltpu.VMEM((16,), jnp.int32),
            pltpu.VMEM((PP,), jnp.int32),
            pltpu.VMEM((PP,), jnp.float32),
            pltpu.VMEM((K * T,), jnp.int32),
            pltpu.VMEM((16, D), jnp.float32),
            pltpu.VMEM((16, D), jnp.float32),
            pltpu.SemaphoreType.DMA,
            pltpu.SemaphoreType.DMA,
            pltpu.SemaphoreType.DMA,
            pltpu.SemaphoreType.DMA,
        ],
    )
    combine = pl.kernel(
        _combine_body,
        compiler_params=sc_params,
        out_type=jax.ShapeDtypeStruct((T, D), jnp.float32),
        mesh=mesh,
        scratch_types=[
            pltpu.VMEM((TW,), jnp.int32),
            pltpu.VMEM((TW,), jnp.int32),
            pltpu.VMEM((8, D), jnp.float32),
            pltpu.VMEM((8, D), jnp.float32),
            pltpu.VMEM((8, D), jnp.float32),
            pltpu.VMEM((8, D), jnp.float32),
            pltpu.VMEM((8, D), jnp.float32),
            pltpu.VMEM((8, D), jnp.float32),
            pltpu.SemaphoreType.DMA,
            pltpu.SemaphoreType.DMA,
            pltpu.SemaphoreType.DMA,
            pltpu.SemaphoreType.DMA,
            pltpu.SemaphoreType.DMA,
            pltpu.SemaphoreType.DMA,
            pltpu.SemaphoreType.DMA,
            pltpu.SemaphoreType.DMA,
        ],
    )
    return dispatch, combine


def _cast_body(a_ref, o_ref):
    o_ref[...] = a_ref[...].astype(jnp.bfloat16)


def _make_cast(n, r, c):
    return pl.pallas_call(
        _cast_body,
        grid=(n,),
        in_specs=[pl.BlockSpec((1, r, c), lambda i: (i, 0, 0))],
        out_specs=pl.BlockSpec((1, r, c), lambda i: (i, 0, 0)),
        out_shape=jax.ShapeDtypeStruct((n, r, c), jnp.bfloat16),
    )


_cast_w1 = _make_cast(32, FF // 2, D)
_cast_w2 = _make_cast(16, D // 2, FF)
_cast_sw1 = _make_cast(8, 2 * SFF // 8, D)
_cast_sw2 = _make_cast(8, D // 8, SFF)


def _routing_metadata(cntf):
    cnt = cntf[0].astype(jnp.int32)
    pc = ((cnt + BT - 1) // BT) * BT
    cum = jnp.cumsum(pc)
    base = jnp.concatenate(
        [jnp.zeros((1,), jnp.int32), cum[:-1],
         jnp.zeros((16 - E,), jnp.int32)]).astype(jnp.int32)
    used = (cum[-1] // BT).astype(jnp.int32)
    tidx = jnp.arange(NT, dtype=jnp.int32) * BT
    te = jnp.minimum(
        jnp.sum((tidx[:, None] >= cum[None, :]).astype(jnp.int32), axis=1),
        E - 1)
    sp = jnp.concatenate([te, used[None]]).astype(jnp.int32)
    return base, sp


def kernel(hidden_states, gate_w, w1, w2, shared_w1, shared_w2):
    x = hidden_states.reshape(T, D)
    ints, ws, cntf = _router(x, gate_w)
    base, sp = _routing_metadata(cntf)

    sw1_bf = _cast_sw1(shared_w1.reshape(8, 2 * SFF // 8, D)).reshape(2 * SFF, D)
    sw2_bf = _cast_sw2(shared_w2.reshape(8, D // 8, SFF)).reshape(D, SFF)
    sh = _shared(x, sw1_bf, sw2_bf)

    dispatch, combine = _sc_kernels()
    xs, wsort, slots = dispatch(ints.T, ws.T, base, x)

    w1_bf = _cast_w1(w1.reshape(32, FF // 2, D)).reshape(E, 2 * FF, D)
    w2_bf = _cast_w2(w2.reshape(16, D // 2, FF)).reshape(E, D, FF)
    y = _gmm(sp, xs, w1_bf, w1_bf, w2_bf, wsort.reshape(PP, 1))

    out = combine(slots, y, sh)
    return out.reshape(1, T, D)


# speedup vs baseline: 1.9736x; 1.0592x over previous
import functools

import jax
import jax.numpy as jnp
from jax import lax
from jax.experimental import pallas as pl
from jax.experimental.pallas import tpu as pltpu
from jax.experimental.pallas import tpu_sc as plsc

E = 8
D = 2048
FF = 1408
SFF = 2816
T = 2048
K = 2
BT = 256
NT = 23
PP = NT * BT
NC = 2
NS = 16
NW = NC * NS
RW = PP // NW
TW = T // NW
TBR = 128


def _router_body(x_ref, g_ref, ints_ref, ws_ref, cnt_ref):
    i = pl.program_id(0)

    @pl.when(i == 0)
    def _():
        cnt_ref[...] = jnp.zeros_like(cnt_ref)

    x = x_ref[...]
    logits = lax.dot_general(
        x, g_ref[...], (((1,), (1,)), ((), ())),
        preferred_element_type=jnp.float32)

    tri = (lax.broadcasted_iota(jnp.int32, (E, E), 0)
           <= lax.broadcasted_iota(jnp.int32, (E, E), 1)).astype(jnp.float32)
    iota_e = lax.broadcasted_iota(jnp.int32, (TBR, E), 1).astype(jnp.float32)

    def pick(m):
        r = jnp.max(m, axis=1, keepdims=True)
        hit = (m == r).astype(jnp.float32)
        csum = lax.dot_general(hit, tri, (((1,), (0,)), ((), ())),
                               preferred_element_type=jnp.float32)
        first = hit * (csum == 1.0).astype(jnp.float32)
        e = jnp.sum(first * iota_e, axis=1, keepdims=True)
        return r, first, e

    r1, f1, e1 = pick(logits)
    r2, f2, e2 = pick(logits - f1 * 1e30)
    w_a = 1.0 / (1.0 + jnp.exp(r2 - r1))
    w_b = 1.0 / (1.0 + jnp.exp(r1 - r2))

    cnt = f1 + f2
    lower = (lax.broadcasted_iota(jnp.int32, (TBR, TBR), 1)
             < lax.broadcasted_iota(jnp.int32, (TBR, TBR), 0)).astype(jnp.float32)
    pos_in = lax.dot_general(lower, cnt, (((1,), (0,)), ((), ())),
                             preferred_element_type=jnp.float32)
    prev = cnt_ref[...]
    pos = pos_in + prev
    p1 = jnp.sum(f1 * pos, axis=1, keepdims=True)
    p2 = jnp.sum(f2 * pos, axis=1, keepdims=True)
    cnt_ref[...] = prev + jnp.sum(cnt, axis=0, keepdims=True)

    col4 = lax.broadcasted_iota(jnp.int32, (TBR, 4), 1)
    iv = jnp.where(col4 == 0, e1,
                   jnp.where(col4 == 1, e2, jnp.where(col4 == 2, p1, p2)))
    ints_ref[...] = iv.astype(jnp.int32)
    col2 = lax.broadcasted_iota(jnp.int32, (TBR, 2), 1)
    ws_ref[...] = jnp.where(col2 == 0, w_a, w_b)


_router = pl.pallas_call(
    _router_body,
    grid=(T // TBR,),
    in_specs=[
        pl.BlockSpec((TBR, D), lambda i: (i, 0)),
        pl.BlockSpec((E, D), lambda i: (0, 0)),
    ],
    out_specs=[
        pl.BlockSpec((TBR, 4), lambda i: (i, 0)),
        pl.BlockSpec((TBR, 2), lambda i: (i, 0)),
        pl.BlockSpec((1, E), lambda i: (0, 0)),
    ],
    out_shape=[
        jax.ShapeDtypeStruct((T, 4), jnp.int32),
        jax.ShapeDtypeStruct((T, 2), jnp.float32),
        jax.ShapeDtypeStruct((1, E), jnp.float32),
    ],
)


_CHUNKS = [(j * 16, 16) for j in range(11)] + [(176, 8)]


def _dispatch_body(ints_hbm, ws_hbm, base_hbm, x_hbm,
                   xs_hbm, wso_hbm, slot_hbm,
                   ints_v, ws_v, base_v, tok_v, wv, slots_v,
                   rows_a, rows_b, gsem_a, gsem_b, wsem_a, wsem_b):
    wid = lax.axis_index("s") * NC + lax.axis_index("c")
    pltpu.sync_copy(ints_hbm, ints_v)
    pltpu.sync_copy(ws_hbm, ws_v)
    pltpu.sync_copy(base_hbm, base_v)

    def zero_body(j, _):
        tok_v[pl.ds(j * 16, 16)] = jnp.zeros((16,), jnp.int32)
        wv[pl.ds(j * 16, 16)] = jnp.zeros((16,), jnp.float32)
        return 0

    lax.fori_loop(0, PP // 16, zero_body, 0)

    def scat_body(c, _):
        t0 = c * 16
        tok = lax.iota(jnp.int32, 16) + t0
        for k in range(K):
            e = ints_v[k, pl.ds(t0, 16)]
            p = ints_v[K + k, pl.ds(t0, 16)]
            w = ws_v[k, pl.ds(t0, 16)]
            slot = plsc.load_gather(base_v, [e]) + p
            plsc.store_scatter(tok_v, [slot], tok)
            plsc.store_scatter(wv, [slot], w)
            slots_v[pl.ds(k * T + t0, 16)] = slot
        return 0

    lax.fori_loop(0, T // 16, scat_body, 0)

    pltpu.sync_copy(wv.at[pl.ds(wid * RW, RW)], wso_hbm.at[pl.ds(wid * RW, RW)])

    @pl.when(wid == 0)
    def _():
        pltpu.sync_copy(slots_v, slot_hbm)

    bufs = (rows_a, rows_b)
    gsems = (gsem_a, gsem_b)
    wsems = (wsem_a, wsem_b)
    for j, (off, n) in enumerate(_CHUNKS):
        b = j % 2
        woff = min(off, RW - 16)
        live = jnp.max(jnp.abs(wv[pl.ds(wid * RW + woff, 16)])) > 0.0

        @pl.when(live)
        def _(b=b, off=off, n=n):
            idx = tok_v.at[pl.ds(wid * RW + off, n)]
            pltpu.async_copy(x_hbm.at[idx], bufs[b].at[pl.ds(0, n)],
                             gsems[b]).wait()
            pltpu.async_copy(bufs[b].at[pl.ds(0, n)],
                             xs_hbm.at[pl.ds(wid * RW + off, n)],
                             wsems[b]).wait()


def _gmm_body(sp_ref, xs_ref, w1g_ref, w1u_ref, w2_ref, ws_ref, y_ref):
    i = pl.program_id(0)

    @pl.when(i < sp_ref[NT])
    def _():
        a = xs_ref[...].astype(jnp.bfloat16)
        g = lax.dot_general(a, w1g_ref[0], (((1,), (1,)), ((), ())),
                            preferred_element_type=jnp.float32)
        u = lax.dot_general(a, w1u_ref[0], (((1,), (1,)), ((), ())),
                            preferred_element_type=jnp.float32)
        act = g * jax.nn.sigmoid(g) * u
        y = lax.dot_general(act, w2_ref[0], (((1,), (1,)), ((), ())),
                            preferred_element_type=jnp.float32)
        y_ref[...] = y * ws_ref[...]


_gmm = pl.pallas_call(
    _gmm_body,
    grid_spec=pltpu.PrefetchScalarGridSpec(
        num_scalar_prefetch=1,
        grid=(NT,),
        in_specs=[
            pl.BlockSpec((BT, D), lambda i, sp: (i, 0)),
            pl.BlockSpec((1, FF, D), lambda i, sp: (sp[i], 0, 0)),
            pl.BlockSpec((1, FF, D), lambda i, sp: (sp[i], 1, 0)),
            pl.BlockSpec((1, D, FF), lambda i, sp: (sp[i], 0, 0)),
            pl.BlockSpec((BT, 1), lambda i, sp: (i, 0)),
        ],
        out_specs=pl.BlockSpec((BT, D), lambda i, sp: (i, 0)),
    ),
    out_shape=jax.ShapeDtypeStruct((PP, D), jnp.float32),
)


def _shared_body(x_ref, w1_ref, w2_ref, o_ref):
    a = x_ref[...].astype(jnp.bfloat16)
    gu = lax.dot_general(a, w1_ref[...], (((1,), (1,)), ((), ())),
                         preferred_element_type=jnp.float32)
    g = gu[:, :SFF]
    u = gu[:, SFF:]
    act = (g * jax.nn.sigmoid(g) * u).astype(jnp.bfloat16)
    o_ref[...] = lax.dot_general(act, w2_ref[...], (((1,), (1,)), ((), ())),
                                 preferred_element_type=jnp.float32)


_shared = pl.pallas_call(
    _shared_body,
    grid=(T // BT,),
    in_specs=[
        pl.BlockSpec((BT, D), lambda i: (i, 0)),
        pl.BlockSpec((2 * SFF, D), lambda i: (0, 0)),
        pl.BlockSpec((D, SFF), lambda i: (0, 0)),
    ],
    out_specs=pl.BlockSpec((BT, D), lambda i: (i, 0)),
    out_shape=jax.ShapeDtypeStruct((T, D), jnp.float32),
)


def _combine_body(slot_hbm, y_hbm, sh_hbm, out_hbm,
                  s1_v, s2_v, y1a, y1b, y2a, y2b, sha, shb,
                  g1a, g1b, g2a, g2b, gha, ghb, wsa, wsb):
    wid = lax.axis_index("s") * NC + lax.axis_index("c")
    t0 = wid * TW
    pltpu.sync_copy(slot_hbm.at[pl.ds(t0, TW)], s1_v)
    pltpu.sync_copy(slot_hbm.at[pl.ds(T + t0, TW)], s2_v)

    y1 = (y1a, y1b)
    y2 = (y2a, y2b)
    sh = (sha, shb)
    g1 = (g1a, g1b)
    g2 = (g2a, g2b)
    gh = (gha, ghb)
    wsems = (wsa, wsb)
    nch = TW // 8

    def issue(c):
        b = c % 2
        tc0 = c * 8
        return (pltpu.async_copy(y_hbm.at[s1_v.at[pl.ds(tc0, 8)]], y1[b], g1[b]),
                pltpu.async_copy(y_hbm.at[s2_v.at[pl.ds(tc0, 8)]], y2[b], g2[b]),
                pltpu.async_copy(sh_hbm.at[pl.ds(t0 + tc0, 8)], sh[b], gh[b]))

    cps = {0: issue(0)}
    wcp = {}
    for c in range(nch):
        b = c % 2
        if c + 1 < nch:
            if c >= 1:
                wcp[c - 1].wait()
            cps[c + 1] = issue(c + 1)
        for cp in cps.pop(c):
            cp.wait()
        for r in range(8):
            def add_body(j, _, r=r, b=b):
                s0 = pl.ds(j * 32, 16)
                s1 = pl.ds(j * 32 + 16, 16)
                sh[b][r, s0] = sh[b][r, s0] + y1[b][r, s0] + y2[b][r, s0]
                sh[b][r, s1] = sh[b][r, s1] + y1[b][r, s1] + y2[b][r, s1]
                return 0

            lax.fori_loop(0, D // 32, add_body, 0)
        wcp[c] = pltpu.async_copy(sh[b], out_hbm.at[pl.ds(t0 + c * 8, 8)],
                                  wsems[b])
    wcp[nch - 2].wait()
    wcp[nch - 1].wait()


@functools.cache
def _sc_kernels():
    mesh = plsc.VectorSubcoreMesh(
        core_axis_name="c", subcore_axis_name="s",
        num_cores=NC, num_subcores=NS)
    sc_params = pltpu.CompilerParams(needs_layout_passes=False)
    dispatch = pl.kernel(
        _dispatch_body,
        compiler_params=sc_params,
        out_type=(
            jax.ShapeDtypeStruct((PP, D), jnp.float32),
            jax.ShapeDtypeStruct((PP,), jnp.float32),
            jax.ShapeDtypeStruct((K * T,), jnp.int32),
        ),
        mesh=mesh,
        scratch_types=[
            pltpu.VMEM((4, T), jnp.int32),
            pltpu.VMEM((K, T), jnp.float32),
            pltpu.VMEM((16,), jnp.int32),
            pltpu.VMEM((PP,), jnp.int32),
            pltpu.VMEM((PP,), jnp.float32),
            pltpu.VMEM((K * T,), jnp.int32),
            pltpu.VMEM((16, D), jnp.float32),
            pltpu.VMEM((16, D), jnp.float32),
            pltpu.SemaphoreType.DMA,
            pltpu.SemaphoreType.DMA,
            pltpu.SemaphoreType.DMA,
            pltpu.SemaphoreType.DMA,
        ],
    )
    combine = pl.kernel(
        _combine_body,
        compiler_params=sc_params,
        out_type=jax.ShapeDtypeStruct((T, D), jnp.float32),
        mesh=mesh,
        scratch_types=[
            pltpu.VMEM((TW,), jnp.int32),
            pltpu.VMEM((TW,), jnp.int32),
            pltpu.VMEM((8, D), jnp.float32),
            pltpu.VMEM((8, D), jnp.float32),
            pltpu.VMEM((8, D), jnp.float32),
            pltpu.VMEM((8, D), jnp.float32),
            pltpu.VMEM((8, D), jnp.float32),
            pltpu.VMEM((8, D), jnp.float32),
            pltpu.SemaphoreType.DMA,
            pltpu.SemaphoreType.DMA,
            pltpu.SemaphoreType.DMA,
            pltpu.SemaphoreType.DMA,
            pltpu.SemaphoreType.DMA,
            pltpu.SemaphoreType.DMA,
            pltpu.SemaphoreType.DMA,
            pltpu.SemaphoreType.DMA,
        ],
    )
    return dispatch, combine


def _cast_body(a_ref, o_ref):
    o_ref[...] = a_ref[...].astype(jnp.bfloat16)


def _make_cast(n, r, c):
    return pl.pallas_call(
        _cast_body,
        grid=(n,),
        in_specs=[pl.BlockSpec((1, r, c), lambda i: (i, 0, 0))],
        out_specs=pl.BlockSpec((1, r, c), lambda i: (i, 0, 0)),
        out_shape=jax.ShapeDtypeStruct((n, r, c), jnp.bfloat16),
    )


_cast_w1 = _make_cast(32, FF // 2, D)
_cast_sw1 = _make_cast(8, 2 * SFF // 8, D)
_cast_sw2 = _make_cast(8, D // 8, SFF)


def _routing_metadata(cntf):
    cnt = cntf[0].astype(jnp.int32)
    pc = ((cnt + BT - 1) // BT) * BT
    cum = jnp.cumsum(pc)
    base = jnp.concatenate(
        [jnp.zeros((1,), jnp.int32), cum[:-1],
         jnp.zeros((16 - E,), jnp.int32)]).astype(jnp.int32)
    used = (cum[-1] // BT).astype(jnp.int32)
    tidx = jnp.arange(NT, dtype=jnp.int32) * BT
    te = jnp.minimum(
        jnp.sum((tidx[:, None] >= cum[None, :]).astype(jnp.int32), axis=1),
        E - 1)
    sp = jnp.concatenate([te, used[None]]).astype(jnp.int32)
    return base, sp


def kernel(hidden_states, gate_w, w1, w2, shared_w1, shared_w2):
    x = hidden_states.reshape(T, D)
    ints, ws, cntf = _router(x, gate_w)
    base, sp = _routing_metadata(cntf)

    sw1_bf = _cast_sw1(shared_w1.reshape(8, 2 * SFF // 8, D)).reshape(2 * SFF, D)
    sw2_bf = _cast_sw2(shared_w2.reshape(8, D // 8, SFF)).reshape(D, SFF)
    sh = _shared(x, sw1_bf, sw2_bf)

    dispatch, combine = _sc_kernels()
    xs, wsort, slots = dispatch(ints.T, ws.T, base, x)

    w1_bf = _cast_w1(w1.reshape(32, FF // 2, D)).reshape(E, 2 * FF, D)
    y = _gmm(sp, xs, w1_bf, w1_bf, w2, wsort.reshape(PP, 1))

    out = combine(slots, y, sh)
    return out.reshape(1, T, D)
